# Initial kernel scaffold; baseline (speedup 1.0000x reference)
import jax
import jax.numpy as jnp
from jax.experimental import pallas as pl


def kernel(mlm_logits):
    raise NotImplementedError("write your pallas kernel here")


# speedup vs baseline: 1.2598x; 1.2598x over previous
import functools

import jax
import jax.numpy as jnp
from jax import lax
from jax.experimental import pallas as pl
from jax.experimental.pallas import tpu as pltpu
from jax.experimental.pallas import tpu_sc as plsc

_B = 128
_N = 512
_L = 16
_NC = 2
_NS = 16
_NW = _NC * _NS
_RPW = _B // _NW
_CHUNKS = _N // _L

_mesh = plsc.VectorSubcoreMesh(
    core_axis_name="c", subcore_axis_name="s", num_cores=_NC, num_subcores=_NS
)


@functools.partial(
    pl.kernel,
    out_type=jax.ShapeDtypeStruct((_B, 2, _N), jnp.float32),
    mesh=_mesh,
    scratch_types=[
        pltpu.VMEM((_RPW, _N), jnp.float32),
        pltpu.VMEM((_RPW, 2, _N), jnp.float32),
    ],
    compiler_params=pltpu.CompilerParams(needs_layout_passes=False),
)
def _compact(x_hbm, out_hbm, rows_v, out_v):
    wid = lax.axis_index("s") * _NC + lax.axis_index("c")
    base = wid * _RPW
    pltpu.sync_copy(x_hbm.at[pl.ds(base, _RPW)], rows_v)

    zf = jnp.zeros((_L,), jnp.float32)
    iota = lax.iota(jnp.int32, _L)
    for r in range(_RPW):
        for c in range(_CHUNKS):
            out_v[r, 0, pl.ds(c * _L, _L)] = zf
            out_v[r, 1, pl.ds(c * _L, _L)] = zf

    for r in range(_RPW):
        r_splat = jnp.full((_L,), r, jnp.int32)
        plane0 = jnp.zeros((_L,), jnp.int32)
        plane1 = jnp.ones((_L,), jnp.int32)
        n_vec = jnp.zeros((_L,), jnp.int32)
        for c in range(_CHUNKS):
            v = rows_v[r, pl.ds(c * _L, _L)]
            m = v != 0.0
            mi = jnp.where(m, 1, 0).astype(jnp.int32)
            dest = plsc.cumsum(mi) + n_vec - 1
            idx_f = (iota + c * _L).astype(jnp.float32)
            plsc.store_scatter(out_v, [r_splat, plane0, dest], idx_f, mask=m)
            plsc.store_scatter(out_v, [r_splat, plane1, dest], v, mask=m)
            n_vec = n_vec + plsc.all_reduce_population_count(m)

    pltpu.sync_copy(out_v, out_hbm.at[pl.ds(base, _RPW)])


def kernel(mlm_logits):
    return _compact(mlm_logits)


# speedup vs baseline: 1.2904x; 1.0243x over previous
import functools

import jax
import jax.numpy as jnp
from jax import lax
from jax.experimental import pallas as pl
from jax.experimental.pallas import tpu as pltpu
from jax.experimental.pallas import tpu_sc as plsc

_B = 128
_N = 512
_L = 16
_NC = 2
_NS = 16
_NW = _NC * _NS
_RPW = _B // _NW
_CHUNKS = _N // _L

_mesh = plsc.VectorSubcoreMesh(
    core_axis_name="c", subcore_axis_name="s", num_cores=_NC, num_subcores=_NS
)


@functools.partial(
    pl.kernel,
    out_type=jax.ShapeDtypeStruct((_B, 2, _N), jnp.float32),
    mesh=_mesh,
    scratch_types=[
        pltpu.VMEM((_RPW, _N), jnp.float32),
        pltpu.VMEM((_RPW, 2, _N), jnp.float32),
        pltpu.VMEM((_N,), jnp.float32),
        pltpu.SemaphoreType.DMA,
    ],
    compiler_params=pltpu.CompilerParams(needs_layout_passes=False),
)
def _compact(x_hbm, out_hbm, rows_v, out_v, idxp_v, sem):
    wid = lax.axis_index("s") * _NC + lax.axis_index("c")
    base = wid * _RPW
    in_cp = pltpu.make_async_copy(x_hbm.at[pl.ds(base, _RPW)], rows_v, sem)
    in_cp.start()

    iota = lax.iota(jnp.int32, _L)
    for c in range(_CHUNKS):
        idxp_v[pl.ds(c * _L, _L)] = (iota + c * _L).astype(jnp.float32)
    in_cp.wait()

    zf = jnp.zeros((_L,), jnp.float32)
    for r in range(_RPW):
        anyz = jnp.zeros((_L,), jnp.bool_)
        for c in range(_CHUNKS):
            anyz = anyz | (rows_v[r, pl.ds(c * _L, _L)] == 0.0)
        haszero = jnp.any(anyz)

        @pl.when(jnp.logical_not(haszero))
        def _fast(r=r):
            for c in range(_CHUNKS):
                sl = pl.ds(c * _L, _L)
                out_v[r, 0, sl] = idxp_v[sl]
                out_v[r, 1, sl] = rows_v[r, sl]

        @pl.when(haszero)
        def _general(r=r):
            r_splat = jnp.full((_L,), r, jnp.int32)
            plane0 = jnp.zeros((_L,), jnp.int32)
            plane1 = jnp.ones((_L,), jnp.int32)
            for c in range(_CHUNKS):
                out_v[r, 0, pl.ds(c * _L, _L)] = zf
                out_v[r, 1, pl.ds(c * _L, _L)] = zf
            n_off = jnp.full((_L,), -1, jnp.int32)
            for c in range(_CHUNKS):
                sl = pl.ds(c * _L, _L)
                v = rows_v[r, sl]
                m = v != 0.0
                dest = plsc.cumsum(m.astype(jnp.int32)) + n_off
                plsc.store_scatter(
                    out_v, [r_splat, plane0, dest], idxp_v[sl], mask=m
                )
                plsc.store_scatter(out_v, [r_splat, plane1, dest], v, mask=m)
                n_off = n_off + plsc.all_reduce_population_count(m)

    pltpu.sync_copy(out_v, out_hbm.at[pl.ds(base, _RPW)])


def kernel(mlm_logits):
    return _compact(mlm_logits)


# speedup vs baseline: 1.5848x; 1.2281x over previous
import functools

import jax
import jax.numpy as jnp
from jax import lax
from jax.experimental import pallas as pl
from jax.experimental.pallas import tpu as pltpu
from jax.experimental.pallas import tpu_sc as plsc

_B = 128
_N = 512
_L = 16
_NC = 2
_NS = 16
_NW = _NC * _NS
_RPW = _B // _NW
_CHUNKS = _N // _L

_mesh = plsc.VectorSubcoreMesh(
    core_axis_name="c", subcore_axis_name="s", num_cores=_NC, num_subcores=_NS
)


@functools.partial(
    pl.kernel,
    out_type=jax.ShapeDtypeStruct((_B, 2, _N), jnp.float32),
    mesh=_mesh,
    scratch_types=[
        pltpu.VMEM((_RPW, _N), jnp.float32),
        pltpu.VMEM((_RPW, 2, _N), jnp.float32),
        pltpu.VMEM((_N,), jnp.float32),
        pltpu.SemaphoreType.DMA,
    ],
    compiler_params=pltpu.CompilerParams(needs_layout_passes=False),
)
def _compact(x_hbm, out_hbm, rows_v, out_v, idxp_v, sem):
    wid = lax.axis_index("s") * _NC + lax.axis_index("c")
    base = wid * _RPW
    in_cp = pltpu.make_async_copy(x_hbm.at[pl.ds(base, _RPW)], rows_v, sem)
    in_cp.start()

    iota = lax.iota(jnp.int32, _L)

    def iota_body(c, _):
        idxp_v[pl.ds(c * _L, _L)] = (iota + c * _L).astype(jnp.float32)
        return 0

    lax.fori_loop(0, _CHUNKS, iota_body, 0, unroll=4)
    in_cp.wait()

    zf = jnp.zeros((_L,), jnp.float32)

    def row_body(r, _):
        def det_body(c, anyz):
            return anyz | (rows_v[r, pl.ds(c * _L, _L)] == 0.0)

        anyz = lax.fori_loop(
            0, _CHUNKS, det_body, jnp.zeros((_L,), jnp.bool_), unroll=4
        )
        haszero = jnp.any(anyz)

        @pl.when(jnp.logical_not(haszero))
        def _fast():
            def cp_body(c, _):
                sl = pl.ds(c * _L, _L)
                out_v[r, 0, sl] = idxp_v[sl]
                out_v[r, 1, sl] = rows_v[r, sl]
                return 0

            lax.fori_loop(0, _CHUNKS, cp_body, 0, unroll=4)

        @pl.when(haszero)
        def _general():
            r_splat = jnp.full((_L,), r, jnp.int32)
            plane0 = jnp.zeros((_L,), jnp.int32)
            plane1 = jnp.ones((_L,), jnp.int32)

            def zero_body(c, _):
                out_v[r, 0, pl.ds(c * _L, _L)] = zf
                out_v[r, 1, pl.ds(c * _L, _L)] = zf
                return 0

            lax.fori_loop(0, _CHUNKS, zero_body, 0, unroll=4)

            def pack_body(c, n_off):
                sl = pl.ds(c * _L, _L)
                v = rows_v[r, sl]
                m = v != 0.0
                dest = plsc.cumsum(m.astype(jnp.int32)) + n_off
                plsc.store_scatter(
                    out_v, [r_splat, plane0, dest], idxp_v[sl], mask=m
                )
                plsc.store_scatter(out_v, [r_splat, plane1, dest], v, mask=m)
                return n_off + plsc.all_reduce_population_count(m)

            lax.fori_loop(
                0, _CHUNKS, pack_body, jnp.full((_L,), -1, jnp.int32)
            )

        return 0

    lax.fori_loop(0, _RPW, row_body, 0)
    pltpu.sync_copy(out_v, out_hbm.at[pl.ds(base, _RPW)])


def kernel(mlm_logits):
    return _compact(mlm_logits)
